# baseline (speedup 1.0000x reference)
import jax
import jax.numpy as jnp
from jax.experimental import pallas as pl
from jax.experimental.pallas import tpu as pltpu

_K = 8
_LOSS_WEIGHT = 0.02
_BIG = 3.0e38
_IBIG = 2**30
_NS = 3
_RH = 64


def _norm_kernel(feat_ref, out_ref):
    f = feat_ref[...]
    n2 = jnp.sum(f * f, axis=1, keepdims=True)
    n = jnp.maximum(jnp.sqrt(n2), 1e-12)
    out_ref[...] = (f / n).astype(jnp.bfloat16)


def _knn_kernel(cb_row_ref, cb_t_ref, sq_row_ref, sq_t_ref, featn_ref, acc_ref,
                keep_ref):
    i = pl.program_id(0)
    R = cb_row_ref.shape[0]
    n = cb_t_ref.shape[1]

    dot = jax.lax.dot(
        cb_row_ref[...], cb_t_ref[...], preferred_element_type=jnp.float32
    )
    d2 = (sq_row_ref[...] + sq_t_ref[...]) - 2.0 * dot
    d2 = jnp.maximum(d2, 0.0)

    v0_parts, v9_parts = [], []
    for rh in range(R // _RH):
        r0 = rh * _RH
        sv = [jnp.full((_RH, 128), _BIG, jnp.float32) for _ in range(_NS)]
        for t in range(n // 128):
            x = d2[r0 : r0 + _RH, t * 128 : (t + 1) * 128]
            for s in range(_NS):
                lo = jnp.minimum(sv[s], x)
                if s + 1 < _NS:
                    x = jnp.maximum(sv[s], x)
                sv[s] = lo
        cand = jnp.concatenate(sv, axis=1)
        pidx = jax.lax.broadcasted_iota(jnp.int32, cand.shape, 1)
        v0 = v9 = None
        for k in range(_K + 1):
            m = jnp.min(cand, axis=1, keepdims=True)
            if k == 0:
                v0 = m
            if k == _K:
                v9 = m
            else:
                hit = cand == m
                pm = jnp.min(jnp.where(hit, pidx, _IBIG), axis=1, keepdims=True)
                cand = jnp.where(hit & (pidx == pm), _BIG, cand)
        v0_parts.append(v0)
        v9_parts.append(v9)
    v0 = jnp.concatenate(v0_parts, axis=0)
    v9 = jnp.concatenate(v9_parts, axis=0)

    cols = jax.lax.broadcasted_iota(jnp.int32, (R, n), 1)
    eq9 = d2 == v9
    j0 = jnp.min(jnp.where(d2 == v0, cols, _IBIG), axis=1, keepdims=True)
    j9 = jnp.min(jnp.where(eq9, cols, _IBIG), axis=1, keepdims=True)
    ub = (d2 < v9) | (eq9 & (cols <= j9))
    keep = jnp.where(ub & (cols != j0), 1.0, 0.0)
    cnt = jnp.sum(keep, axis=1, keepdims=True)
    bad = jnp.sum(jnp.abs(cnt - float(_K))) != 0.0
    keep_ref[...] = keep

    @pl.when(bad)
    def _fallback():
        w = d2
        kp = jnp.zeros((R, n), jnp.float32)
        for k in range(_K + 1):
            m = jnp.min(w, axis=1, keepdims=True)
            hit = w == m
            jm = jnp.min(jnp.where(hit, cols, _IBIG), axis=1, keepdims=True)
            pos = hit & (cols == jm)
            if k > 0:
                kp = kp + pos.astype(jnp.float32)
            w = jnp.where(pos, _BIG, w)
        keep_ref[...] = kp

    s = jax.lax.dot(
        keep_ref[...].astype(jnp.bfloat16),
        featn_ref[...],
        preferred_element_type=jnp.float32,
    )
    frow = featn_ref[pl.ds(i * R, R), :].astype(jnp.float32)
    c = jnp.sum(frow * s)

    @pl.when(i == 0)
    def _():
        acc_ref[...] = jnp.zeros_like(acc_ref)

    acc_ref[...] += c[None, None]


def kernel(feat_3d_list, spatial_coords_list):
    feat = feat_3d_list
    coords = spatial_coords_list
    n, dfeat = feat.shape

    featn = pl.pallas_call(
        _norm_kernel,
        grid=(n // 512,),
        in_specs=[pl.BlockSpec((512, dfeat), lambda i: (i, 0))],
        out_specs=pl.BlockSpec((512, dfeat), lambda i: (i, 0)),
        out_shape=jax.ShapeDtypeStruct((n, dfeat), jnp.bfloat16),
    )(feat)

    R = 128
    cb = coords.astype(jnp.bfloat16)
    cb_t = cb.T
    sq = jnp.sum(coords * coords, axis=-1)
    sq_col = sq[:, None]
    sq_row_b = sq[None, :]

    acc = pl.pallas_call(
        _knn_kernel,
        grid=(n // R,),
        in_specs=[
            pl.BlockSpec((R, 3), lambda i: (i, 0)),
            pl.BlockSpec((3, n), lambda i: (0, 0)),
            pl.BlockSpec((R, 1), lambda i: (i, 0)),
            pl.BlockSpec((1, n), lambda i: (0, 0)),
            pl.BlockSpec((n, dfeat), lambda i: (0, 0)),
        ],
        out_specs=pl.BlockSpec((1, 1), lambda i: (0, 0)),
        out_shape=jax.ShapeDtypeStruct((1, 1), jnp.float32),
        scratch_shapes=[pltpu.VMEM((R, n), jnp.float32)],
        compiler_params=pltpu.CompilerParams(
            dimension_semantics=("arbitrary",),
        ),
    )(cb, cb_t, sq_col, sq_row_b, featn)

    total = acc[0, 0]
    return _LOSS_WEIGHT * (1.0 - total / (n * _K))


# speedup vs baseline: 3.6562x; 3.6562x over previous
import jax
import jax.numpy as jnp
from jax.experimental import pallas as pl
from jax.experimental.pallas import tpu as pltpu

_K = 8
_LOSS_WEIGHT = 0.02
_BIG = 3.0e38
_IBIG = 2**30
_NS = 3
_RH = 64
_ZEPS = 1e-13


def _norm_kernel(feat_ref, out_ref):
    f = feat_ref[...]
    n2 = jnp.sum(f * f, axis=1, keepdims=True)
    n = jnp.maximum(jnp.sqrt(n2), 1e-12)
    out_ref[...] = (f / n).astype(jnp.bfloat16)


def _knn_kernel(cb_row_ref, cb_t_ref, sq_row_ref, sq_t_ref, zcol_ref,
                featn_ref, acc_ref, keep_ref):
    i = pl.program_id(0)
    R = cb_row_ref.shape[0]
    n = cb_t_ref.shape[1]

    dot = jax.lax.dot(
        cb_row_ref[...], cb_t_ref[...], preferred_element_type=jnp.float32
    )
    d2 = (sq_row_ref[...] + sq_t_ref[...]) - 2.0 * dot
    d2 = jnp.maximum(d2, 0.0)
    key = jnp.where(d2 == 0.0, zcol_ref[...], d2)

    v0_parts, v9_parts = [], []
    for rh in range(R // _RH):
        r0 = rh * _RH
        sv = [jnp.full((_RH, 128), _BIG, jnp.float32) for _ in range(_NS)]
        for t in range(n // 128):
            x = key[r0 : r0 + _RH, t * 128 : (t + 1) * 128]
            for s in range(_NS):
                lo = jnp.minimum(sv[s], x)
                if s + 1 < _NS:
                    x = jnp.maximum(sv[s], x)
                sv[s] = lo
        cand = jnp.concatenate(sv, axis=1)
        pidx = jax.lax.broadcasted_iota(jnp.int32, cand.shape, 1)
        v0 = v9 = None
        for k in range(_K + 1):
            m = jnp.min(cand, axis=1, keepdims=True)
            if k == 0:
                v0 = m
            if k == _K:
                v9 = m
            else:
                hit = cand == m
                pm = jnp.min(jnp.where(hit, pidx, _IBIG), axis=1, keepdims=True)
                cand = jnp.where(hit & (pidx == pm), _BIG, cand)
        v0_parts.append(v0)
        v9_parts.append(v9)
    v0 = jnp.concatenate(v0_parts, axis=0)
    v9 = jnp.concatenate(v9_parts, axis=0)

    keep = jnp.where((key <= v9) & (key != v0), 1.0, 0.0)
    cnt = jnp.sum(keep, axis=1, keepdims=True)
    bad = jnp.sum(jnp.abs(cnt - float(_K))) != 0.0
    keep_ref[...] = keep

    @pl.when(bad)
    def _fallback():
        cols = jax.lax.broadcasted_iota(jnp.int32, (R, n), 1)
        w = key
        kp = jnp.zeros((R, n), jnp.float32)
        for k in range(_K + 1):
            m = jnp.min(w, axis=1, keepdims=True)
            hit = w == m
            jm = jnp.min(jnp.where(hit, cols, _IBIG), axis=1, keepdims=True)
            pos = hit & (cols == jm)
            if k > 0:
                kp = kp + pos.astype(jnp.float32)
            w = jnp.where(pos, _BIG, w)
        keep_ref[...] = kp

    s = jax.lax.dot(
        keep_ref[...].astype(jnp.bfloat16),
        featn_ref[...],
        preferred_element_type=jnp.float32,
    )
    frow = featn_ref[pl.ds(i * R, R), :].astype(jnp.float32)
    c = jnp.sum(frow * s)

    @pl.when(i == 0)
    def _():
        acc_ref[...] = jnp.zeros_like(acc_ref)

    acc_ref[...] += c[None, None]


def kernel(feat_3d_list, spatial_coords_list):
    feat = feat_3d_list
    coords = spatial_coords_list
    n, dfeat = feat.shape

    featn = pl.pallas_call(
        _norm_kernel,
        grid=(n // 512,),
        in_specs=[pl.BlockSpec((512, dfeat), lambda i: (i, 0))],
        out_specs=pl.BlockSpec((512, dfeat), lambda i: (i, 0)),
        out_shape=jax.ShapeDtypeStruct((n, dfeat), jnp.bfloat16),
    )(feat)

    R = 128
    cb = coords.astype(jnp.bfloat16)
    cb_t = cb.T
    sq = jnp.sum(coords * coords, axis=-1)
    sq_col = sq[:, None]
    sq_row_b = sq[None, :]
    zcol = (jnp.arange(n, dtype=jnp.float32) * _ZEPS)[None, :]

    acc = pl.pallas_call(
        _knn_kernel,
        grid=(n // R,),
        in_specs=[
            pl.BlockSpec((R, 3), lambda i: (i, 0)),
            pl.BlockSpec((3, n), lambda i: (0, 0)),
            pl.BlockSpec((R, 1), lambda i: (i, 0)),
            pl.BlockSpec((1, n), lambda i: (0, 0)),
            pl.BlockSpec((1, n), lambda i: (0, 0)),
            pl.BlockSpec((n, dfeat), lambda i: (0, 0)),
        ],
        out_specs=pl.BlockSpec((1, 1), lambda i: (0, 0)),
        out_shape=jax.ShapeDtypeStruct((1, 1), jnp.float32),
        scratch_shapes=[pltpu.VMEM((R, n), jnp.float32)],
        compiler_params=pltpu.CompilerParams(
            dimension_semantics=("arbitrary",),
        ),
    )(cb, cb_t, sq_col, sq_row_b, zcol, featn)

    total = acc[0, 0]
    return _LOSS_WEIGHT * (1.0 - total / (n * _K))
